# baseline (speedup 1.0000x reference)
import functools
import math

import jax
import jax.numpy as jnp
import numpy as np
from jax.experimental import pallas as pl
from jax.experimental.pallas import tpu as pltpu

F32 = jnp.float32
BF16 = jnp.bfloat16
PREC = jax.lax.Precision.HIGHEST
CPREC = jax.lax.Precision.DEFAULT
_CP = dict(compiler_params=pltpu.CompilerParams(
    vmem_limit_bytes=100 * 1024 * 1024))


def _resize_mat(T, S, align_corners):
    if align_corners:
        xs = np.linspace(np.float32(0.0), np.float32(S - 1), T).astype(np.float32)
    else:
        xs = np.maximum(
            (np.arange(T, dtype=np.float32) + np.float32(0.5))
            * np.float32(S / T) - np.float32(0.5), np.float32(0.0))
    x0 = np.clip(np.floor(xs).astype(np.int64), 0, S - 1)
    x1 = np.clip(x0 + 1, 0, S - 1)
    w = (xs - x0.astype(np.float32)).astype(np.float32)
    M = np.zeros((T, S), dtype=np.float32)
    M[np.arange(T), x0] += (1.0 - w)
    M[np.arange(T), x1] += w
    return M


def _pool_mat(T):
    M = np.zeros((T, 2 * T), dtype=np.float32)
    M[np.arange(T), 2 * np.arange(T)] = 0.5
    M[np.arange(T), 2 * np.arange(T) + 1] = 0.5
    return M


_R128 = _resize_mat(128, 256, True)
_R64 = _resize_mat(64, 256, True)
_T224_256 = _resize_mat(224, 256, False)
_T224_128 = _resize_mat(224, 128, False)
_T224_64 = _resize_mat(224, 64, False)
_C128 = (_T224_128 @ _R128).astype(np.float32)
_C64 = (_T224_64 @ _R64).astype(np.float32)
_P112 = _pool_mat(112)
_P56 = _pool_mat(56)


def _resize_body(x_ref, a_ref, b_ref, o_ref):
    x = x_ref[0, 0]
    t = jax.lax.dot(a_ref[...], x, preferred_element_type=F32, precision=PREC)
    o_ref[0, 0] = jax.lax.dot(t, b_ref[...], preferred_element_type=F32,
                              precision=PREC)


def _resize(x, A):
    N, C, S, _ = x.shape
    T = A.shape[0]
    Aj = jnp.asarray(A)
    Bj = jnp.asarray(A.T.copy())
    return pl.pallas_call(
        _resize_body,
        grid=(N, C),
        in_specs=[
            pl.BlockSpec((1, 1, S, S), lambda n, c: (n, c, 0, 0)),
            pl.BlockSpec((T, S), lambda n, c: (0, 0)),
            pl.BlockSpec((S, T), lambda n, c: (0, 0)),
        ],
        out_specs=pl.BlockSpec((1, 1, T, T), lambda n, c: (n, c, 0, 0)),
        out_shape=jax.ShapeDtypeStruct((N, C, T, T), F32),
        **_CP,
    )(x, Aj, Bj)


def _halo_rows(x, rows):
    N, H, Wp, C = x.shape
    Ht = H // rows
    z = jnp.zeros((N, 1, Wp, C), x.dtype)
    if rows == 1:
        tops = z
        bots = z
    else:
        tops = jnp.concatenate([z, x[:, Ht - 1:H - 1:Ht]], axis=1)
        bots = jnp.concatenate([x[:, Ht:H:Ht], z], axis=1)
    return jnp.stack([tops, bots], axis=2)


def _conv_body(xm_ref, xb_ref, w_ref, b_ref, o_ref, *, Ht, W, Cin, Cout):
    full = jnp.concatenate(
        [xb_ref[0, 0, 0][None], xm_ref[0], xb_ref[0, 0, 1][None]],
        axis=0)
    acc = jnp.zeros((Ht * W, Cout), dtype=F32)
    for dy in range(3):
        for dx in range(3):
            a2 = full[dy:dy + Ht, 7 + dx:7 + dx + W, :].reshape(Ht * W, Cin)
            acc = acc + jax.lax.dot(
                a2, w_ref[dy * 3 + dx], preferred_element_type=F32,
                precision=CPREC)
    y = jnp.maximum(acc + b_ref[...], 0.0).reshape(Ht, W, Cout)
    o_ref[0, :, 8:W + 8, :] = y
    o_ref[0, :, 0:8, :] = jnp.zeros((Ht, 8, Cout), dtype=F32)
    o_ref[0, :, W + 8:W + 10, :] = jnp.zeros((Ht, 2, Cout), dtype=F32)


def _conv(x, w, b, *, rows):
    N, H, Wp, Cin = x.shape
    W = Wp - 10
    Ht = H // rows
    Cout = w.shape[0]
    wt = jnp.transpose(w, (2, 3, 1, 0)).reshape(9, Cin, Cout)
    xb = _halo_rows(x, rows)
    body = functools.partial(_conv_body, Ht=Ht, W=W, Cin=Cin, Cout=Cout)
    return pl.pallas_call(
        body,
        grid=(N, rows),
        in_specs=[
            pl.BlockSpec((1, Ht, Wp, Cin), lambda n, r: (n, r, 0, 0)),
            pl.BlockSpec((1, 1, 2, Wp, Cin), lambda n, r: (n, r, 0, 0, 0)),
            pl.BlockSpec((9, Cin, Cout), lambda n, r: (0, 0, 0)),
            pl.BlockSpec((1, Cout), lambda n, r: (0, 0)),
        ],
        out_specs=pl.BlockSpec((1, Ht, Wp, Cout), lambda n, r: (n, r, 0, 0)),
        out_shape=jax.ShapeDtypeStruct((N, H, Wp, Cout), F32),
        compiler_params=pltpu.CompilerParams(
            dimension_semantics=("parallel", "parallel"),
            vmem_limit_bytes=100 * 1024 * 1024),
    )(x, xb, wt, b.reshape(1, Cout))


def _convp_body(xm_ref, xb_ref, w_ref, b_ref, o_ref, *, Ht, W, Cin, Cout):
    Wp = W + 10
    full = jnp.concatenate(
        [xb_ref[0, 0, 0][None], xm_ref[0], xb_ref[0, 0, 1][None]],
        axis=0)
    a3 = jnp.concatenate(
        [full[0:Ht], full[1:Ht + 1], full[2:Ht + 2]], axis=-1)
    out = jax.lax.dot(a3.reshape(Ht * Wp, 3 * Cin), w_ref[...],
                      preferred_element_type=F32, precision=CPREC)
    out = out.reshape(Ht, Wp, 3 * Cout)
    y = (out[:, 7:W + 7, 0:Cout] + out[:, 8:W + 8, Cout:2 * Cout]
         + out[:, 9:W + 9, 2 * Cout:3 * Cout])
    y = jnp.maximum(y + b_ref[0][None, None, :], 0.0)
    o_ref[0, :, 8:W + 8, :] = y
    o_ref[0, :, 0:8, :] = jnp.zeros((Ht, 8, Cout), dtype=F32)
    o_ref[0, :, W + 8:W + 10, :] = jnp.zeros((Ht, 2, Cout), dtype=F32)


def _convp(x, w, b, *, rows):
    N, H, Wp, Cin = x.shape
    W = Wp - 10
    Ht = H // rows
    Cout = w.shape[0]
    wt = jnp.transpose(w, (2, 1, 3, 0)).reshape(3 * Cin, 3 * Cout)
    xb = _halo_rows(x, rows)
    body = functools.partial(_convp_body, Ht=Ht, W=W, Cin=Cin, Cout=Cout)
    return pl.pallas_call(
        body,
        grid=(N, rows),
        in_specs=[
            pl.BlockSpec((1, Ht, Wp, Cin), lambda n, r: (n, r, 0, 0)),
            pl.BlockSpec((1, 1, 2, Wp, Cin), lambda n, r: (n, r, 0, 0, 0)),
            pl.BlockSpec((3 * Cin, 3 * Cout), lambda n, r: (0, 0)),
            pl.BlockSpec((1, Cout), lambda n, r: (0, 0)),
        ],
        out_specs=pl.BlockSpec((1, Ht, Wp, Cout), lambda n, r: (n, r, 0, 0)),
        out_shape=jax.ShapeDtypeStruct((N, H, Wp, Cout), F32),
        compiler_params=pltpu.CompilerParams(
            dimension_semantics=("parallel", "parallel"),
            vmem_limit_bytes=100 * 1024 * 1024),
    )(x, xb, wt, b.reshape(1, Cout))


def _conv0_body(xm_ref, xb_ref, w_ref, b_ref, o_ref, *, Ht, W, Cout):
    full = jnp.concatenate(
        [xb_ref[0, 0, 0][None], xm_ref[0], xb_ref[0, 0, 1][None]],
        axis=0)
    taps = [full[dy:dy + Ht, 7 + dx:7 + dx + W, :]
            for dy in range(3) for dx in range(3)]
    a27 = jnp.concatenate(taps, axis=-1).reshape(Ht * W, 27)
    acc = jax.lax.dot(a27, w_ref[...], preferred_element_type=F32,
                      precision=CPREC)
    y = jnp.maximum(acc + b_ref[...], 0.0).reshape(Ht, W, Cout)
    o_ref[0, :, 8:W + 8, :] = y
    o_ref[0, :, 0:8, :] = jnp.zeros((Ht, 8, Cout), dtype=F32)
    o_ref[0, :, W + 8:W + 10, :] = jnp.zeros((Ht, 2, Cout), dtype=F32)


def _conv0(x, w, b, *, rows):
    N, H, Wp, _ = x.shape
    W = Wp - 10
    Ht = H // rows
    Cout = w.shape[0]
    wt = jnp.transpose(w, (2, 3, 1, 0)).reshape(27, Cout)
    xb = _halo_rows(x, rows)
    body = functools.partial(_conv0_body, Ht=Ht, W=W, Cout=Cout)
    return pl.pallas_call(
        body,
        grid=(N, rows),
        in_specs=[
            pl.BlockSpec((1, Ht, Wp, 3), lambda n, r: (n, r, 0, 0)),
            pl.BlockSpec((1, 1, 2, Wp, 3), lambda n, r: (n, r, 0, 0, 0)),
            pl.BlockSpec((27, Cout), lambda n, r: (0, 0)),
            pl.BlockSpec((1, Cout), lambda n, r: (0, 0)),
        ],
        out_specs=pl.BlockSpec((1, Ht, Wp, Cout), lambda n, r: (n, r, 0, 0)),
        out_shape=jax.ShapeDtypeStruct((N, H, Wp, Cout), F32),
        compiler_params=pltpu.CompilerParams(
            dimension_semantics=("parallel", "parallel"),
            vmem_limit_bytes=100 * 1024 * 1024),
    )(x, xb, wt, b.reshape(1, Cout))


def _pool_body(x_ref, o_ref, *, Ht, W, C):
    a = x_ref[0, :, 8:W + 8, :]
    a = a.reshape(Ht, 2, W, C)
    a = jnp.max(a, axis=1)
    a = a.reshape(Ht, W // 2, 2, C)
    a = jnp.max(a, axis=2)
    w = W // 2
    o_ref[0, :, 8:w + 8, :] = a
    o_ref[0, :, 0:8, :] = jnp.zeros((Ht, 8, C), dtype=F32)
    o_ref[0, :, w + 8:w + 10, :] = jnp.zeros((Ht, 2, C), dtype=F32)


def _maxpool(x, *, rows):
    N, H, Wp, C = x.shape
    W = Wp - 10
    h, w = H // 2, W // 2
    Ht = h // rows
    body = functools.partial(_pool_body, Ht=Ht, W=W, C=C)
    return pl.pallas_call(
        body,
        grid=(N, rows),
        in_specs=[pl.BlockSpec((1, 2 * Ht, Wp, C), lambda n, r: (n, r, 0, 0))],
        out_specs=pl.BlockSpec((1, Ht, w + 10, C), lambda n, r: (n, r, 0, 0)),
        out_shape=jax.ShapeDtypeStruct((N, h, w + 10, C), F32),
        compiler_params=pltpu.CompilerParams(
            dimension_semantics=("parallel", "parallel"),
            vmem_limit_bytes=100 * 1024 * 1024),
    )(x)


def _gm_body(x_ref, p1_ref, p2_ref, g1_ref, g2_ref):
    es = []
    for n in range(2):
        e = ((x_ref[n, 0] - x_ref[n + 2, 0]) ** 2
             + (x_ref[n, 1] - x_ref[n + 2, 1]) ** 2
             + (x_ref[n, 2] - x_ref[n + 2, 2]) ** 2) / 3.0
        es.append(e)
    mn = jnp.minimum(jnp.min(es[0]), jnp.min(es[1]))
    mx = jnp.maximum(jnp.max(es[0]), jnp.max(es[1]))
    scale = 1.0 / (mx - mn)
    p1 = p1_ref[...]
    p2 = p2_ref[...]
    for n in range(2):
        g = (es[n] - mn) * scale
        t = jax.lax.dot(p1, g, preferred_element_type=F32, precision=PREC)
        g112 = jax.lax.dot(t, p1.T, preferred_element_type=F32,
                           precision=PREC)
        g1_ref[0, n] = g112
        t2 = jax.lax.dot(p2, g112, preferred_element_type=F32, precision=PREC)
        g2_ref[0, n] = jax.lax.dot(t2, p2.T, preferred_element_type=F32,
                                   precision=PREC)


def _gm_maps(xp):
    p1 = jnp.asarray(_P112)
    p2 = jnp.asarray(_P56)
    return pl.pallas_call(
        _gm_body,
        grid=(3,),
        in_specs=[
            pl.BlockSpec((4, 3, 224, 224), lambda i: (i, 0, 0, 0)),
            pl.BlockSpec((112, 224), lambda i: (0, 0)),
            pl.BlockSpec((56, 112), lambda i: (0, 0)),
        ],
        out_specs=[
            pl.BlockSpec((1, 2, 112, 112), lambda i: (i, 0, 0, 0)),
            pl.BlockSpec((1, 2, 56, 56), lambda i: (i, 0, 0, 0)),
        ],
        out_shape=[
            jax.ShapeDtypeStruct((3, 2, 112, 112), F32),
            jax.ShapeDtypeStruct((3, 2, 56, 56), F32),
        ],
        compiler_params=pltpu.CompilerParams(
            dimension_semantics=("parallel",),
            vmem_limit_bytes=100 * 1024 * 1024),
    )(xp, p1, p2)


def _make_stats(U, gm=None, align=False, rows=1):
    N, H, Wp, C = U.shape
    W = Wp - 10
    Ht = H // rows

    if align:
        def body(x_ref, y_ref, o_ref, v_ref):
            x = x_ref[0, :, 8:W + 8, :].astype(F32)
            y = y_ref[0, :, 8:W + 8, :].astype(F32)
            d = x - y
            o_ref[0, 0, 0] = jnp.sum(d * d)
            cx = (jax.lax.broadcasted_iota(jnp.int32, (H, W, C), 1)
                  .astype(F32) / 14.0 - 1.0)
            cy = (jax.lax.broadcasted_iota(jnp.int32, (H, W, C), 0)
                  .astype(F32) / 14.0 - 1.0)
            v_ref[0, 0] = jnp.sum(x, axis=(0, 1))
            v_ref[0, 1] = jnp.sum(cx * x, axis=(0, 1))
            v_ref[0, 2] = jnp.sum(cy * x, axis=(0, 1))
            v_ref[0, 3] = jnp.sum(y, axis=(0, 1))
            v_ref[0, 4] = jnp.sum(cx * y, axis=(0, 1))
            v_ref[0, 5] = jnp.sum(cy * y, axis=(0, 1))

        out_specs = [
            pl.BlockSpec((1, 1, 2), lambda s, r: (s, 0, 0),
                         memory_space=pltpu.SMEM),
            pl.BlockSpec((1, 8, C), lambda s, r: (s, 0, 0)),
        ]
        out_shape = [
            jax.ShapeDtypeStruct((6, 1, 2), F32),
            jax.ShapeDtypeStruct((6, 8, C), F32),
        ]
        extra_in = []
    elif gm is not None:
        def body(x_ref, y_ref, g_ref, o_ref):
            r = pl.program_id(1)

            @pl.when(r == 0)
            def _():
                o_ref[0, 0, 0] = 0.0
                o_ref[0, 0, 1] = 0.0
            d = (x_ref[0, :, 8:W + 8, :].astype(F32)
                 - y_ref[0, :, 8:W + 8, :].astype(F32))
            d2 = d * d
            g = g_ref[0, 0][:, :, None]
            o_ref[0, 0, 0] += jnp.sum(d2)
            o_ref[0, 0, 1] += jnp.sum((g * g) * d2)

        out_specs = [
            pl.BlockSpec((1, 1, 2), lambda s, r: (s, 0, 0),
                         memory_space=pltpu.SMEM)]
        out_shape = [jax.ShapeDtypeStruct((6, 1, 2), F32)]
        extra_in = [
            pl.BlockSpec((1, 1, Ht, W),
                         lambda s, r: (s // 2, s % 2, r, 0))]
    else:
        def body(x_ref, y_ref, o_ref):
            r = pl.program_id(1)

            @pl.when(r == 0)
            def _():
                o_ref[0, 0, 0] = 0.0
            d = (x_ref[0, :, 8:W + 8, :].astype(F32)
                 - y_ref[0, :, 8:W + 8, :].astype(F32))
            o_ref[0, 0, 0] += jnp.sum(d * d)

        out_specs = [
            pl.BlockSpec((1, 1, 2), lambda s, r: (s, 0, 0),
                         memory_space=pltpu.SMEM)]
        out_shape = [jax.ShapeDtypeStruct((6, 1, 2), F32)]
        extra_in = []

    in_specs = [
        pl.BlockSpec((1, Ht, Wp, C),
                     lambda s, r: (4 * (s // 2) + s % 2, r, 0, 0)),
        pl.BlockSpec((1, Ht, Wp, C),
                     lambda s, r: (4 * (s // 2) + s % 2 + 2, r, 0, 0)),
    ] + extra_in

    args = [U, U] + ([gm] if gm is not None else [])
    return pl.pallas_call(
        body,
        grid=(6, rows),
        in_specs=in_specs,
        out_specs=out_specs if len(out_specs) > 1 else out_specs[0],
        out_shape=out_shape if len(out_shape) > 1 else out_shape[0],
        compiler_params=pltpu.CompilerParams(
            dimension_semantics=("parallel", "arbitrary"),
            vmem_limit_bytes=100 * 1024 * 1024),
    )(*args)


def _gram_body(x0_ref, x1_ref, y0_ref, y1_ref, o_ref, acc_ref, *,
               C, W, HWt, rows):
    r = pl.program_id(1)

    @pl.when(r == 0)
    def _():
        acc_ref[...] = jnp.zeros(acc_ref.shape, acc_ref.dtype)

    def blocks(a_ref, b_ref):
        a = a_ref[0, :, 8:W + 8, :].reshape(HWt, C)
        b = b_ref[0, :, 8:W + 8, :].reshape(HWt, C)
        dn = (((0,), (0,)), ((), ()))
        aa = jax.lax.dot_general(a, a, dn, preferred_element_type=F32,
                                 precision=CPREC)
        ab = jax.lax.dot_general(a, b, dn, preferred_element_type=F32,
                                 precision=CPREC)
        bb = jax.lax.dot_general(b, b, dn, preferred_element_type=F32,
                                 precision=CPREC)
        return aa, ab, bb

    xa, xb, xc = blocks(x0_ref, x1_ref)
    ya, yb, yc = blocks(y0_ref, y1_ref)
    acc_ref[0] += xa - ya
    acc_ref[1] += xb - yb
    acc_ref[2] += xc - yc

    @pl.when(r == rows - 1)
    def _():
        d0 = acc_ref[0]
        d1 = acc_ref[1]
        d2 = acc_ref[2]
        o_ref[0, 0, 0] = (jnp.sum(d0 * d0) + 2.0 * jnp.sum(d1 * d1)
                          + jnp.sum(d2 * d2))


def _gram_diff(U, rows):
    N, H, Wp, C = U.shape
    W = Wp - 10
    Ht = H // rows
    body = functools.partial(_gram_body, C=C, W=W, HWt=Ht * W, rows=rows)

    def mk(img_off):
        return pl.BlockSpec(
            (1, Ht, Wp, C), lambda i, r, o=img_off: (4 * i + o, r, 0, 0))

    return pl.pallas_call(
        body,
        grid=(3, rows),
        in_specs=[mk(0), mk(1), mk(2), mk(3)],
        out_specs=pl.BlockSpec((1, 1, 2), lambda i, r: (i, 0, 0),
                               memory_space=pltpu.SMEM),
        out_shape=jax.ShapeDtypeStruct((3, 1, 2), F32),
        scratch_shapes=[pltpu.VMEM((3, C, C), F32)],
        compiler_params=pltpu.CompilerParams(
            dimension_semantics=("parallel", "arbitrary"),
            vmem_limit_bytes=100 * 1024 * 1024),
    )(U, U, U, U)


def _sl1(d):
    a = jnp.abs(d)
    return jnp.where(a < 1.0, 0.5 * a * a, a - 0.5)


def _level_body(dp_ref, tp_ref, m_ref, o_ref, *, H, W):
    q = dp_ref[0]
    p = tp_ref[0]

    def grads(z):
        gy = (z[0:H, 0:W] + 2.0 * z[0:H, 1:W + 1] + z[0:H, 2:W + 2]
              - z[2:H + 2, 0:W] - 2.0 * z[2:H + 2, 1:W + 1]
              - z[2:H + 2, 2:W + 2])
        gx = (z[0:H, 0:W] + 2.0 * z[1:H + 1, 0:W] + z[2:H + 2, 0:W]
              - z[0:H, 2:W + 2] - 2.0 * z[1:H + 1, 2:W + 2]
              - z[2:H + 2, 2:W + 2])
        return gy, gx

    gyf, gxf = grads(q)
    gyr, gxr = grads(p)
    o_ref[0, 0, 0] = jnp.sum(jnp.abs(gyr - gyf)) + jnp.sum(jnp.abs(gxr - gxf))
    o_ref[0, 0, 1] = jnp.sum(gyf * gyr)
    o_ref[0, 0, 2] = jnp.sum(gxf * gxr)
    o_ref[0, 0, 3] = jnp.sum(gyf * gyf)
    o_ref[0, 0, 4] = jnp.sum(gxf * gxf)
    o_ref[0, 0, 5] = jnp.sum(gyr * gyr)
    o_ref[0, 0, 6] = jnp.sum(gxr * gxr)
    m = m_ref[0]
    o_ref[0, 0, 7] = jnp.sum(_sl1(q[1:H + 1, 1:W + 1] - p[1:H + 1, 1:W + 1]) * m)
    o_ref[0, 0, 8] = jnp.sum(m)


def _level_stats(dpad, tpad, m):
    N, Hp, Wp = dpad.shape
    H, W = Hp - 2, Wp - 2
    body = functools.partial(_level_body, H=H, W=W)
    return pl.pallas_call(
        body,
        grid=(N,),
        in_specs=[
            pl.BlockSpec((1, Hp, Wp), lambda n: (n, 0, 0)),
            pl.BlockSpec((1, Hp, Wp), lambda n: (n, 0, 0)),
            pl.BlockSpec((1, H, W), lambda n: (n, 0, 0)),
        ],
        out_specs=pl.BlockSpec((1, 1, 16), lambda n: (n, 0, 0),
                               memory_space=pltpu.SMEM),
        out_shape=jax.ShapeDtypeStruct((N, 1, 16), F32),
        compiler_params=pltpu.CompilerParams(
            dimension_semantics=("parallel",),
            vmem_limit_bytes=100 * 1024 * 1024),
    )(dpad, tpad, m)


def _warp_body(a_ref, b_ref, o_ref):
    o_ref[0, 0, 0] = jnp.sum(_sl1(a_ref[0] - b_ref[0]))


def _warp_sl1_sum(a, b):
    N = a.shape[0]
    S = a.shape[2]
    return pl.pallas_call(
        _warp_body,
        grid=(N,),
        in_specs=[
            pl.BlockSpec((1, 3, S, S), lambda n: (n, 0, 0, 0)),
            pl.BlockSpec((1, 3, S, S), lambda n: (n, 0, 0, 0)),
        ],
        out_specs=pl.BlockSpec((1, 1, 2), lambda n: (n, 0, 0),
                               memory_space=pltpu.SMEM),
        out_shape=jax.ShapeDtypeStruct((N, 1, 2), F32),
        compiler_params=pltpu.CompilerParams(
            dimension_semantics=("parallel",),
            vmem_limit_bytes=100 * 1024 * 1024),
    )(a, b)


def kernel(depth_0, depth_1, depth_2, target_level_0, target_level_1,
           target_level_2, mask_level_0, mask_level_1, mask_level_2,
           warp_view_0, warp_view_1, warp_view_2, imgs,
           vgg_w0, vgg_b0, vgg_w2, vgg_b2, vgg_w5, vgg_b5, vgg_w7, vgg_b7,
           vgg_w10, vgg_b10, vgg_w12, vgg_b12, vgg_w14, vgg_b14,
           vgg_w17, vgg_b17, vgg_w19, vgg_b19, vgg_w21, vgg_b21):
    depths = (depth_0, depth_1, depth_2)
    targets = (target_level_0, target_level_1, target_level_2)
    masks = (mask_level_0, mask_level_1, mask_level_2)
    warps = (warp_view_0, warp_view_1, warp_view_2)

    level_terms = []
    for l in range(3):
        dpad = jnp.pad(depths[l], ((0, 0), (1, 1), (1, 1)))
        tpad = jnp.pad(targets[l], ((0, 0), (1, 1), (1, 1)))
        st = _level_stats(dpad, tpad, masks[l].astype(F32))[:, 0, :]
        H = depths[l].shape[1]
        W = depths[l].shape[2]
        grad_loss = jnp.sum(st[:, 0]) / (2.0 * 2.0 * H * W)
        cosines = []
        for n in range(2):
            cosines.append(st[n, 1] / (jnp.sqrt(st[n, 3]) * jnp.sqrt(st[n, 5])))
            cosines.append(st[n, 2] / (jnp.sqrt(st[n, 4]) * jnp.sqrt(st[n, 6])))
        n_loss = 1.0 - (cosines[0] + cosines[1] + cosines[2] + cosines[3]) / 4.0
        l1 = jnp.sum(st[:, 7]) / jnp.maximum(jnp.sum(st[:, 8]), 1.0)
        level_terms.append((grad_loss + n_loss + l1) * (2.0 ** (1 - l)))

    ref = imgs[:, 0]
    rr1 = _resize(ref, _R128)
    rr2 = _resize(ref, _R64)
    x224 = {
        2: _resize(ref, _C64),
        1: _resize(ref, _C128),
        0: _resize(ref, _T224_256),
    }
    y224 = {
        2: _resize(warps[2], _T224_64),
        1: _resize(warps[1], _T224_128),
        0: _resize(warps[0], _T224_256),
    }
    xp = jnp.concatenate([x224[2], y224[2], x224[1], y224[1],
                          x224[0], y224[0]], axis=0)

    gm112, gm56 = _gm_maps(xp)

    xn = jnp.pad(jnp.transpose(xp, (0, 2, 3, 1)),
                 ((0, 0), (0, 0), (8, 2), (0, 0)))

    a = _convp(xn, vgg_w0, vgg_b0, rows=7)
    u0 = _convp(a, vgg_w2, vgg_b2, rows=7)
    a = _maxpool(u0, rows=7)
    a = _convp(a, vgg_w5, vgg_b5, rows=2)
    u1 = _convp(a, vgg_w7, vgg_b7, rows=2)
    a = _maxpool(u1, rows=4)
    a = _conv(a, vgg_w10, vgg_b10, rows=2)
    a = _conv(a, vgg_w12, vgg_b12, rows=2)
    u2 = _conv(a, vgg_w14, vgg_b14, rows=2)
    a = _maxpool(u2, rows=2)
    a = _conv(a, vgg_w17, vgg_b17, rows=1)
    a = _conv(a, vgg_w19, vgg_b19, rows=1)
    u3 = _conv(a, vgg_w21, vgg_b21, rows=1)

    s0 = _make_stats(u0, rows=8)[:, 0, :]
    s1 = _make_stats(u1, gm=gm112, rows=2)[:, 0, :]
    s2 = _make_stats(u2, gm=gm56)[:, 0, :]
    s3, v3 = _make_stats(u3, align=True)
    s3 = s3[:, 0, :]
    g1 = _gram_diff(u1, rows=4)[:, 0, :]
    g2 = _gram_diff(u2, rows=2)[:, 0, :]
    g3 = _gram_diff(u3, rows=1)[:, 0, :]

    wsum = {
        0: _warp_sl1_sum(ref, warps[0])[:, 0, :],
        1: _warp_sl1_sum(rr1, warps[1])[:, 0, :],
        2: _warp_sl1_sum(rr2, warps[2])[:, 0, :],
    }

    loss = level_terms[0] + level_terms[1] + level_terms[2]
    sizes = {0: 256, 1: 128, 2: 64}
    cdims = [(64, 224), (128, 112), (256, 56), (512, 28)]
    for i, l in enumerate((2, 1, 0)):
        wl = 2.0 ** (1 - l)
        S = sizes[l]
        loss = loss + wl * (wsum[l][0, 0] + wsum[l][1, 0]) / (2.0 * 3 * S * S)

        V = 0.0
        C, HH = cdims[0]
        V = V + (s0[2 * i, 0] + s0[2 * i + 1, 0]) / (2.0 * C * HH * HH)
        for b, sb, gb, wgt in ((1, s1, g1, 1000.0 / (224.0 * 224.0)),
                               (2, s2, g2, 1000.0 / (112.0 * 112.0))):
            C, HH = cdims[b]
            denom = 2.0 * C * HH * HH
            V = V + wgt * (sb[2 * i, 1] + sb[2 * i + 1, 1]) / denom
            V = V + gb[i, 0] / (denom * denom * (2.0 * C) ** 2)
            V = V + (sb[2 * i, 0] + sb[2 * i + 1, 0]) / denom
        C, HH = cdims[3]
        denom = 2.0 * C * HH * HH
        sxg = jnp.concatenate([v3[2 * i, 0], v3[2 * i + 1, 0]]) + 1e-07
        uxg = jnp.concatenate([v3[2 * i, 1], v3[2 * i + 1, 1]]) + 1e-07
        vxg = jnp.concatenate([v3[2 * i, 2], v3[2 * i + 1, 2]]) + 1e-07
        syt = jnp.concatenate([v3[2 * i, 3], v3[2 * i + 1, 3]]) + 1e-07
        uyt = jnp.concatenate([v3[2 * i, 4], v3[2 * i + 1, 4]]) + 1e-07
        vyt = jnp.concatenate([v3[2 * i, 5], v3[2 * i + 1, 5]]) + 1e-07
        cuk = uyt / syt
        cvk = vyt / syt
        cukp = uxg / sxg
        cvkp = vxg / sxg
        align = (jnp.sum((cuk - cukp) ** 2) + jnp.sum((cvk - cvkp) ** 2)) \
            / (2.0 * cuk.shape[0])
        V = V + align
        V = V + g3[i, 0] / (denom * denom * (2.0 * C) ** 2)
        V = V + (s3[2 * i, 0] + s3[2 * i + 1, 0]) / denom
        loss = loss + wl * V

    return loss


# speedup vs baseline: 1.0134x; 1.0134x over previous
import functools
import math

import jax
import jax.numpy as jnp
import numpy as np
from jax.experimental import pallas as pl
from jax.experimental.pallas import tpu as pltpu

F32 = jnp.float32
BF16 = jnp.bfloat16
PREC = jax.lax.Precision.HIGHEST
CPREC = jax.lax.Precision.DEFAULT
_CP = dict(compiler_params=pltpu.CompilerParams(
    vmem_limit_bytes=100 * 1024 * 1024))


def _resize_mat(T, S, align_corners):
    if align_corners:
        xs = np.linspace(np.float32(0.0), np.float32(S - 1), T).astype(np.float32)
    else:
        xs = np.maximum(
            (np.arange(T, dtype=np.float32) + np.float32(0.5))
            * np.float32(S / T) - np.float32(0.5), np.float32(0.0))
    x0 = np.clip(np.floor(xs).astype(np.int64), 0, S - 1)
    x1 = np.clip(x0 + 1, 0, S - 1)
    w = (xs - x0.astype(np.float32)).astype(np.float32)
    M = np.zeros((T, S), dtype=np.float32)
    M[np.arange(T), x0] += (1.0 - w)
    M[np.arange(T), x1] += w
    return M


def _pool_mat(T):
    M = np.zeros((T, 2 * T), dtype=np.float32)
    M[np.arange(T), 2 * np.arange(T)] = 0.5
    M[np.arange(T), 2 * np.arange(T) + 1] = 0.5
    return M


_R128 = _resize_mat(128, 256, True)
_R64 = _resize_mat(64, 256, True)
_T224_256 = _resize_mat(224, 256, False)
_T224_128 = _resize_mat(224, 128, False)
_T224_64 = _resize_mat(224, 64, False)
_C128 = (_T224_128 @ _R128).astype(np.float32)
_C64 = (_T224_64 @ _R64).astype(np.float32)
_P112 = _pool_mat(112)
_P56 = _pool_mat(56)


def _resize_body(x_ref, a_ref, b_ref, o_ref):
    x = x_ref[0, 0]
    t = jax.lax.dot(a_ref[...], x, preferred_element_type=F32, precision=PREC)
    o_ref[0, 0] = jax.lax.dot(t, b_ref[...], preferred_element_type=F32,
                              precision=PREC)


def _resize(x, A):
    N, C, S, _ = x.shape
    T = A.shape[0]
    Aj = jnp.asarray(A)
    Bj = jnp.asarray(A.T.copy())
    return pl.pallas_call(
        _resize_body,
        grid=(N, C),
        in_specs=[
            pl.BlockSpec((1, 1, S, S), lambda n, c: (n, c, 0, 0)),
            pl.BlockSpec((T, S), lambda n, c: (0, 0)),
            pl.BlockSpec((S, T), lambda n, c: (0, 0)),
        ],
        out_specs=pl.BlockSpec((1, 1, T, T), lambda n, c: (n, c, 0, 0)),
        out_shape=jax.ShapeDtypeStruct((N, C, T, T), F32),
        **_CP,
    )(x, Aj, Bj)


def _halo_rows(x, rows):
    N, H, Wp, C = x.shape
    Ht = H // rows
    z = jnp.zeros((N, 1, Wp, C), x.dtype)
    if rows == 1:
        tops = z
        bots = z
    else:
        tops = jnp.concatenate([z, x[:, Ht - 1:H - 1:Ht]], axis=1)
        bots = jnp.concatenate([x[:, Ht:H:Ht], z], axis=1)
    return jnp.stack([tops, bots], axis=2)


def _conv_body(xm_ref, xb_ref, w_ref, b_ref, o_ref, *, Ht, W, Cin, Cout):
    full = jnp.concatenate(
        [xb_ref[0, 0, 0][None], xm_ref[0], xb_ref[0, 0, 1][None]],
        axis=0)
    acc = jnp.zeros((Ht * W, Cout), dtype=F32)
    for dy in range(3):
        for dx in range(3):
            a2 = full[dy:dy + Ht, dx:dx + W, :].reshape(Ht * W, Cin)
            acc = acc + jax.lax.dot(
                a2, w_ref[dy * 3 + dx], preferred_element_type=F32,
                precision=CPREC)
    y = jnp.maximum(acc + b_ref[...], 0.0).reshape(Ht, W, Cout)
    o_ref[0, :, 1:W + 1, :] = y
    o_ref[0, :, 0:1, :] = jnp.zeros((Ht, 1, Cout), dtype=F32)
    o_ref[0, :, W + 1:W + 2, :] = jnp.zeros((Ht, 1, Cout), dtype=F32)


def _conv(x, w, b, *, rows):
    N, H, Wp, Cin = x.shape
    W = Wp - 2
    Ht = H // rows
    Cout = w.shape[0]
    wt = jnp.transpose(w, (2, 3, 1, 0)).reshape(9, Cin, Cout)
    xb = _halo_rows(x, rows)
    body = functools.partial(_conv_body, Ht=Ht, W=W, Cin=Cin, Cout=Cout)
    return pl.pallas_call(
        body,
        grid=(N, rows),
        in_specs=[
            pl.BlockSpec((1, Ht, Wp, Cin), lambda n, r: (n, r, 0, 0)),
            pl.BlockSpec((1, 1, 2, Wp, Cin), lambda n, r: (n, r, 0, 0, 0)),
            pl.BlockSpec((9, Cin, Cout), lambda n, r: (0, 0, 0)),
            pl.BlockSpec((1, Cout), lambda n, r: (0, 0)),
        ],
        out_specs=pl.BlockSpec((1, Ht, Wp, Cout), lambda n, r: (n, r, 0, 0)),
        out_shape=jax.ShapeDtypeStruct((N, H, Wp, Cout), F32),
        compiler_params=pltpu.CompilerParams(
            dimension_semantics=("parallel", "parallel"),
            vmem_limit_bytes=100 * 1024 * 1024),
    )(x, xb, wt, b.reshape(1, Cout))


def _convp_body(xm_ref, xb_ref, w_ref, b_ref, o_ref, *, Ht, W, Cin, Cout):
    Wp = W + 2
    full = jnp.concatenate(
        [xb_ref[0, 0, 0][None], xm_ref[0], xb_ref[0, 0, 1][None]],
        axis=0)
    a3 = jnp.concatenate(
        [full[0:Ht], full[1:Ht + 1], full[2:Ht + 2]], axis=-1)
    out = jax.lax.dot(a3.reshape(Ht * Wp, 3 * Cin), w_ref[...],
                      preferred_element_type=F32, precision=CPREC)
    out = out.reshape(Ht, Wp, 3 * Cout)
    y = (out[:, 0:W, 0:Cout] + out[:, 1:W + 1, Cout:2 * Cout]
         + out[:, 2:W + 2, 2 * Cout:3 * Cout])
    y = jnp.maximum(y + b_ref[0][None, None, :], 0.0)
    o_ref[0, :, 1:W + 1, :] = y
    o_ref[0, :, 0:1, :] = jnp.zeros((Ht, 1, Cout), dtype=F32)
    o_ref[0, :, W + 1:W + 2, :] = jnp.zeros((Ht, 1, Cout), dtype=F32)


def _convp(x, w, b, *, rows):
    N, H, Wp, Cin = x.shape
    W = Wp - 2
    Ht = H // rows
    Cout = w.shape[0]
    wt = jnp.transpose(w, (2, 1, 3, 0)).reshape(3 * Cin, 3 * Cout)
    xb = _halo_rows(x, rows)
    body = functools.partial(_convp_body, Ht=Ht, W=W, Cin=Cin, Cout=Cout)
    return pl.pallas_call(
        body,
        grid=(N, rows),
        in_specs=[
            pl.BlockSpec((1, Ht, Wp, Cin), lambda n, r: (n, r, 0, 0)),
            pl.BlockSpec((1, 1, 2, Wp, Cin), lambda n, r: (n, r, 0, 0, 0)),
            pl.BlockSpec((3 * Cin, 3 * Cout), lambda n, r: (0, 0)),
            pl.BlockSpec((1, Cout), lambda n, r: (0, 0)),
        ],
        out_specs=pl.BlockSpec((1, Ht, Wp, Cout), lambda n, r: (n, r, 0, 0)),
        out_shape=jax.ShapeDtypeStruct((N, H, Wp, Cout), F32),
        compiler_params=pltpu.CompilerParams(
            dimension_semantics=("parallel", "parallel"),
            vmem_limit_bytes=100 * 1024 * 1024),
    )(x, xb, wt, b.reshape(1, Cout))


def _conv0_body(xm_ref, xb_ref, w_ref, b_ref, o_ref, *, Ht, W, Cout):
    full = jnp.concatenate(
        [xb_ref[0, 0, 0][None], xm_ref[0], xb_ref[0, 0, 1][None]],
        axis=0)
    taps = [full[dy:dy + Ht, dx:dx + W, :]
            for dy in range(3) for dx in range(3)]
    a27 = jnp.concatenate(taps, axis=-1).reshape(Ht * W, 27)
    acc = jax.lax.dot(a27, w_ref[...], preferred_element_type=F32,
                      precision=CPREC)
    y = jnp.maximum(acc + b_ref[...], 0.0).reshape(Ht, W, Cout)
    o_ref[0, :, 1:W + 1, :] = y
    o_ref[0, :, 0:1, :] = jnp.zeros((Ht, 1, Cout), dtype=F32)
    o_ref[0, :, W + 1:W + 2, :] = jnp.zeros((Ht, 1, Cout), dtype=F32)


def _conv0(x, w, b, *, rows):
    N, H, Wp, _ = x.shape
    W = Wp - 2
    Ht = H // rows
    Cout = w.shape[0]
    wt = jnp.transpose(w, (2, 3, 1, 0)).reshape(27, Cout)
    xb = _halo_rows(x, rows)
    body = functools.partial(_conv0_body, Ht=Ht, W=W, Cout=Cout)
    return pl.pallas_call(
        body,
        grid=(N, rows),
        in_specs=[
            pl.BlockSpec((1, Ht, Wp, 3), lambda n, r: (n, r, 0, 0)),
            pl.BlockSpec((1, 1, 2, Wp, 3), lambda n, r: (n, r, 0, 0, 0)),
            pl.BlockSpec((27, Cout), lambda n, r: (0, 0)),
            pl.BlockSpec((1, Cout), lambda n, r: (0, 0)),
        ],
        out_specs=pl.BlockSpec((1, Ht, Wp, Cout), lambda n, r: (n, r, 0, 0)),
        out_shape=jax.ShapeDtypeStruct((N, H, Wp, Cout), F32),
        compiler_params=pltpu.CompilerParams(
            dimension_semantics=("parallel", "parallel"),
            vmem_limit_bytes=100 * 1024 * 1024),
    )(x, xb, wt, b.reshape(1, Cout))


def _pool_body(x_ref, o_ref, *, Ht, W, C):
    a = x_ref[0, :, 1:W + 1, :]
    a = a.reshape(Ht, 2, W, C)
    a = jnp.max(a, axis=1)
    a = a.reshape(Ht, W // 2, 2, C)
    a = jnp.max(a, axis=2)
    w = W // 2
    o_ref[0, :, 1:w + 1, :] = a
    o_ref[0, :, 0:1, :] = jnp.zeros((Ht, 1, C), dtype=F32)
    o_ref[0, :, w + 1:w + 2, :] = jnp.zeros((Ht, 1, C), dtype=F32)


def _maxpool(x, *, rows):
    N, H, Wp, C = x.shape
    W = Wp - 2
    h, w = H // 2, W // 2
    Ht = h // rows
    body = functools.partial(_pool_body, Ht=Ht, W=W, C=C)
    return pl.pallas_call(
        body,
        grid=(N, rows),
        in_specs=[pl.BlockSpec((1, 2 * Ht, Wp, C), lambda n, r: (n, r, 0, 0))],
        out_specs=pl.BlockSpec((1, Ht, w + 2, C), lambda n, r: (n, r, 0, 0)),
        out_shape=jax.ShapeDtypeStruct((N, h, w + 2, C), F32),
        compiler_params=pltpu.CompilerParams(
            dimension_semantics=("parallel", "parallel"),
            vmem_limit_bytes=100 * 1024 * 1024),
    )(x)


def _gm_body(x_ref, p1_ref, p2_ref, g1_ref, g2_ref):
    es = []
    for n in range(2):
        e = ((x_ref[n, 0] - x_ref[n + 2, 0]) ** 2
             + (x_ref[n, 1] - x_ref[n + 2, 1]) ** 2
             + (x_ref[n, 2] - x_ref[n + 2, 2]) ** 2) / 3.0
        es.append(e)
    mn = jnp.minimum(jnp.min(es[0]), jnp.min(es[1]))
    mx = jnp.maximum(jnp.max(es[0]), jnp.max(es[1]))
    scale = 1.0 / (mx - mn)
    p1 = p1_ref[...]
    p2 = p2_ref[...]
    for n in range(2):
        g = (es[n] - mn) * scale
        t = jax.lax.dot(p1, g, preferred_element_type=F32, precision=PREC)
        g112 = jax.lax.dot(t, p1.T, preferred_element_type=F32,
                           precision=PREC)
        g1_ref[0, n] = g112
        t2 = jax.lax.dot(p2, g112, preferred_element_type=F32, precision=PREC)
        g2_ref[0, n] = jax.lax.dot(t2, p2.T, preferred_element_type=F32,
                                   precision=PREC)


def _gm_maps(xp):
    p1 = jnp.asarray(_P112)
    p2 = jnp.asarray(_P56)
    return pl.pallas_call(
        _gm_body,
        grid=(3,),
        in_specs=[
            pl.BlockSpec((4, 3, 224, 224), lambda i: (i, 0, 0, 0)),
            pl.BlockSpec((112, 224), lambda i: (0, 0)),
            pl.BlockSpec((56, 112), lambda i: (0, 0)),
        ],
        out_specs=[
            pl.BlockSpec((1, 2, 112, 112), lambda i: (i, 0, 0, 0)),
            pl.BlockSpec((1, 2, 56, 56), lambda i: (i, 0, 0, 0)),
        ],
        out_shape=[
            jax.ShapeDtypeStruct((3, 2, 112, 112), F32),
            jax.ShapeDtypeStruct((3, 2, 56, 56), F32),
        ],
        compiler_params=pltpu.CompilerParams(
            dimension_semantics=("parallel",),
            vmem_limit_bytes=100 * 1024 * 1024),
    )(xp, p1, p2)


def _make_stats(U, gm=None, align=False, rows=1):
    N, H, Wp, C = U.shape
    W = Wp - 2
    Ht = H // rows

    if align:
        def body(x_ref, y_ref, o_ref, v_ref):
            x = x_ref[0, :, 1:W + 1, :].astype(F32)
            y = y_ref[0, :, 1:W + 1, :].astype(F32)
            d = x - y
            o_ref[0, 0, 0] = jnp.sum(d * d)
            cx = (jax.lax.broadcasted_iota(jnp.int32, (H, W, C), 1)
                  .astype(F32) / 14.0 - 1.0)
            cy = (jax.lax.broadcasted_iota(jnp.int32, (H, W, C), 0)
                  .astype(F32) / 14.0 - 1.0)
            v_ref[0, 0] = jnp.sum(x, axis=(0, 1))
            v_ref[0, 1] = jnp.sum(cx * x, axis=(0, 1))
            v_ref[0, 2] = jnp.sum(cy * x, axis=(0, 1))
            v_ref[0, 3] = jnp.sum(y, axis=(0, 1))
            v_ref[0, 4] = jnp.sum(cx * y, axis=(0, 1))
            v_ref[0, 5] = jnp.sum(cy * y, axis=(0, 1))

        out_specs = [
            pl.BlockSpec((1, 1, 2), lambda s, r: (s, 0, 0),
                         memory_space=pltpu.SMEM),
            pl.BlockSpec((1, 8, C), lambda s, r: (s, 0, 0)),
        ]
        out_shape = [
            jax.ShapeDtypeStruct((6, 1, 2), F32),
            jax.ShapeDtypeStruct((6, 8, C), F32),
        ]
        extra_in = []
    elif gm is not None:
        def body(x_ref, y_ref, g_ref, o_ref):
            r = pl.program_id(1)

            @pl.when(r == 0)
            def _():
                o_ref[0, 0, 0] = 0.0
                o_ref[0, 0, 1] = 0.0
            d = (x_ref[0, :, 1:W + 1, :].astype(F32)
                 - y_ref[0, :, 1:W + 1, :].astype(F32))
            d2 = d * d
            g = g_ref[0, 0][:, :, None]
            o_ref[0, 0, 0] += jnp.sum(d2)
            o_ref[0, 0, 1] += jnp.sum((g * g) * d2)

        out_specs = [
            pl.BlockSpec((1, 1, 2), lambda s, r: (s, 0, 0),
                         memory_space=pltpu.SMEM)]
        out_shape = [jax.ShapeDtypeStruct((6, 1, 2), F32)]
        extra_in = [
            pl.BlockSpec((1, 1, Ht, W),
                         lambda s, r: (s // 2, s % 2, r, 0))]
    else:
        def body(x_ref, y_ref, o_ref):
            r = pl.program_id(1)

            @pl.when(r == 0)
            def _():
                o_ref[0, 0, 0] = 0.0
            d = (x_ref[0, :, 1:W + 1, :].astype(F32)
                 - y_ref[0, :, 1:W + 1, :].astype(F32))
            o_ref[0, 0, 0] += jnp.sum(d * d)

        out_specs = [
            pl.BlockSpec((1, 1, 2), lambda s, r: (s, 0, 0),
                         memory_space=pltpu.SMEM)]
        out_shape = [jax.ShapeDtypeStruct((6, 1, 2), F32)]
        extra_in = []

    in_specs = [
        pl.BlockSpec((1, Ht, Wp, C),
                     lambda s, r: (4 * (s // 2) + s % 2, r, 0, 0)),
        pl.BlockSpec((1, Ht, Wp, C),
                     lambda s, r: (4 * (s // 2) + s % 2 + 2, r, 0, 0)),
    ] + extra_in

    args = [U, U] + ([gm] if gm is not None else [])
    return pl.pallas_call(
        body,
        grid=(6, rows),
        in_specs=in_specs,
        out_specs=out_specs if len(out_specs) > 1 else out_specs[0],
        out_shape=out_shape if len(out_shape) > 1 else out_shape[0],
        compiler_params=pltpu.CompilerParams(
            dimension_semantics=("parallel", "arbitrary"),
            vmem_limit_bytes=100 * 1024 * 1024),
    )(*args)


def _gram_body(x0_ref, x1_ref, y0_ref, y1_ref, o_ref, acc_ref, *,
               C, W, HWt, rows):
    r = pl.program_id(1)

    @pl.when(r == 0)
    def _():
        acc_ref[...] = jnp.zeros(acc_ref.shape, acc_ref.dtype)

    def blocks(a_ref, b_ref):
        a = a_ref[0, :, 1:W + 1, :].reshape(HWt, C)
        b = b_ref[0, :, 1:W + 1, :].reshape(HWt, C)
        dn = (((0,), (0,)), ((), ()))
        aa = jax.lax.dot_general(a, a, dn, preferred_element_type=F32,
                                 precision=CPREC)
        ab = jax.lax.dot_general(a, b, dn, preferred_element_type=F32,
                                 precision=CPREC)
        bb = jax.lax.dot_general(b, b, dn, preferred_element_type=F32,
                                 precision=CPREC)
        return aa, ab, bb

    xa, xb, xc = blocks(x0_ref, x1_ref)
    ya, yb, yc = blocks(y0_ref, y1_ref)
    acc_ref[0] += xa - ya
    acc_ref[1] += xb - yb
    acc_ref[2] += xc - yc

    @pl.when(r == rows - 1)
    def _():
        d0 = acc_ref[0]
        d1 = acc_ref[1]
        d2 = acc_ref[2]
        o_ref[0, 0, 0] = (jnp.sum(d0 * d0) + 2.0 * jnp.sum(d1 * d1)
                          + jnp.sum(d2 * d2))


def _gram_diff(U, rows):
    N, H, Wp, C = U.shape
    W = Wp - 2
    Ht = H // rows
    body = functools.partial(_gram_body, C=C, W=W, HWt=Ht * W, rows=rows)

    def mk(img_off):
        return pl.BlockSpec(
            (1, Ht, Wp, C), lambda i, r, o=img_off: (4 * i + o, r, 0, 0))

    return pl.pallas_call(
        body,
        grid=(3, rows),
        in_specs=[mk(0), mk(1), mk(2), mk(3)],
        out_specs=pl.BlockSpec((1, 1, 2), lambda i, r: (i, 0, 0),
                               memory_space=pltpu.SMEM),
        out_shape=jax.ShapeDtypeStruct((3, 1, 2), F32),
        scratch_shapes=[pltpu.VMEM((3, C, C), F32)],
        compiler_params=pltpu.CompilerParams(
            dimension_semantics=("parallel", "arbitrary"),
            vmem_limit_bytes=100 * 1024 * 1024),
    )(U, U, U, U)


def _sl1(d):
    a = jnp.abs(d)
    return jnp.where(a < 1.0, 0.5 * a * a, a - 0.5)


def _level_body(dp_ref, tp_ref, m_ref, o_ref, *, H, W):
    q = dp_ref[0]
    p = tp_ref[0]

    def grads(z):
        gy = (z[0:H, 0:W] + 2.0 * z[0:H, 1:W + 1] + z[0:H, 2:W + 2]
              - z[2:H + 2, 0:W] - 2.0 * z[2:H + 2, 1:W + 1]
              - z[2:H + 2, 2:W + 2])
        gx = (z[0:H, 0:W] + 2.0 * z[1:H + 1, 0:W] + z[2:H + 2, 0:W]
              - z[0:H, 2:W + 2] - 2.0 * z[1:H + 1, 2:W + 2]
              - z[2:H + 2, 2:W + 2])
        return gy, gx

    gyf, gxf = grads(q)
    gyr, gxr = grads(p)
    o_ref[0, 0, 0] = jnp.sum(jnp.abs(gyr - gyf)) + jnp.sum(jnp.abs(gxr - gxf))
    o_ref[0, 0, 1] = jnp.sum(gyf * gyr)
    o_ref[0, 0, 2] = jnp.sum(gxf * gxr)
    o_ref[0, 0, 3] = jnp.sum(gyf * gyf)
    o_ref[0, 0, 4] = jnp.sum(gxf * gxf)
    o_ref[0, 0, 5] = jnp.sum(gyr * gyr)
    o_ref[0, 0, 6] = jnp.sum(gxr * gxr)
    m = m_ref[0]
    o_ref[0, 0, 7] = jnp.sum(_sl1(q[1:H + 1, 1:W + 1] - p[1:H + 1, 1:W + 1]) * m)
    o_ref[0, 0, 8] = jnp.sum(m)


def _level_stats(dpad, tpad, m):
    N, Hp, Wp = dpad.shape
    H, W = Hp - 2, Wp - 2
    body = functools.partial(_level_body, H=H, W=W)
    return pl.pallas_call(
        body,
        grid=(N,),
        in_specs=[
            pl.BlockSpec((1, Hp, Wp), lambda n: (n, 0, 0)),
            pl.BlockSpec((1, Hp, Wp), lambda n: (n, 0, 0)),
            pl.BlockSpec((1, H, W), lambda n: (n, 0, 0)),
        ],
        out_specs=pl.BlockSpec((1, 1, 16), lambda n: (n, 0, 0),
                               memory_space=pltpu.SMEM),
        out_shape=jax.ShapeDtypeStruct((N, 1, 16), F32),
        compiler_params=pltpu.CompilerParams(
            dimension_semantics=("parallel",),
            vmem_limit_bytes=100 * 1024 * 1024),
    )(dpad, tpad, m)


def _warp_body(a_ref, b_ref, o_ref):
    o_ref[0, 0, 0] = jnp.sum(_sl1(a_ref[0] - b_ref[0]))


def _warp_sl1_sum(a, b):
    N = a.shape[0]
    S = a.shape[2]
    return pl.pallas_call(
        _warp_body,
        grid=(N,),
        in_specs=[
            pl.BlockSpec((1, 3, S, S), lambda n: (n, 0, 0, 0)),
            pl.BlockSpec((1, 3, S, S), lambda n: (n, 0, 0, 0)),
        ],
        out_specs=pl.BlockSpec((1, 1, 2), lambda n: (n, 0, 0),
                               memory_space=pltpu.SMEM),
        out_shape=jax.ShapeDtypeStruct((N, 1, 2), F32),
        compiler_params=pltpu.CompilerParams(
            dimension_semantics=("parallel",),
            vmem_limit_bytes=100 * 1024 * 1024),
    )(a, b)


def kernel(depth_0, depth_1, depth_2, target_level_0, target_level_1,
           target_level_2, mask_level_0, mask_level_1, mask_level_2,
           warp_view_0, warp_view_1, warp_view_2, imgs,
           vgg_w0, vgg_b0, vgg_w2, vgg_b2, vgg_w5, vgg_b5, vgg_w7, vgg_b7,
           vgg_w10, vgg_b10, vgg_w12, vgg_b12, vgg_w14, vgg_b14,
           vgg_w17, vgg_b17, vgg_w19, vgg_b19, vgg_w21, vgg_b21):
    depths = (depth_0, depth_1, depth_2)
    targets = (target_level_0, target_level_1, target_level_2)
    masks = (mask_level_0, mask_level_1, mask_level_2)
    warps = (warp_view_0, warp_view_1, warp_view_2)

    level_terms = []
    for l in range(3):
        dpad = jnp.pad(depths[l], ((0, 0), (1, 1), (1, 1)))
        tpad = jnp.pad(targets[l], ((0, 0), (1, 1), (1, 1)))
        st = _level_stats(dpad, tpad, masks[l].astype(F32))[:, 0, :]
        H = depths[l].shape[1]
        W = depths[l].shape[2]
        grad_loss = jnp.sum(st[:, 0]) / (2.0 * 2.0 * H * W)
        cosines = []
        for n in range(2):
            cosines.append(st[n, 1] / (jnp.sqrt(st[n, 3]) * jnp.sqrt(st[n, 5])))
            cosines.append(st[n, 2] / (jnp.sqrt(st[n, 4]) * jnp.sqrt(st[n, 6])))
        n_loss = 1.0 - (cosines[0] + cosines[1] + cosines[2] + cosines[3]) / 4.0
        l1 = jnp.sum(st[:, 7]) / jnp.maximum(jnp.sum(st[:, 8]), 1.0)
        level_terms.append((grad_loss + n_loss + l1) * (2.0 ** (1 - l)))

    ref = imgs[:, 0]
    rr1 = _resize(ref, _R128)
    rr2 = _resize(ref, _R64)
    x224 = {
        2: _resize(ref, _C64),
        1: _resize(ref, _C128),
        0: _resize(ref, _T224_256),
    }
    y224 = {
        2: _resize(warps[2], _T224_64),
        1: _resize(warps[1], _T224_128),
        0: _resize(warps[0], _T224_256),
    }
    xp = jnp.concatenate([x224[2], y224[2], x224[1], y224[1],
                          x224[0], y224[0]], axis=0)

    gm112, gm56 = _gm_maps(xp)

    xn = jnp.pad(jnp.transpose(xp, (0, 2, 3, 1)),
                 ((0, 0), (0, 0), (1, 1), (0, 0)))

    a = _convp(xn, vgg_w0, vgg_b0, rows=7)
    u0 = _convp(a, vgg_w2, vgg_b2, rows=7)
    a = _maxpool(u0, rows=7)
    a = _convp(a, vgg_w5, vgg_b5, rows=2)
    u1 = _convp(a, vgg_w7, vgg_b7, rows=2)
    a = _maxpool(u1, rows=4)
    a = _conv(a, vgg_w10, vgg_b10, rows=2)
    a = _conv(a, vgg_w12, vgg_b12, rows=2)
    u2 = _conv(a, vgg_w14, vgg_b14, rows=2)
    a = _maxpool(u2, rows=2)
    a = _conv(a, vgg_w17, vgg_b17, rows=1)
    a = _conv(a, vgg_w19, vgg_b19, rows=1)
    u3 = _conv(a, vgg_w21, vgg_b21, rows=1)

    s0 = _make_stats(u0, rows=8)[:, 0, :]
    s1 = _make_stats(u1, gm=gm112, rows=2)[:, 0, :]
    s2 = _make_stats(u2, gm=gm56)[:, 0, :]
    s3, v3 = _make_stats(u3, align=True)
    s3 = s3[:, 0, :]
    g1 = _gram_diff(u1, rows=4)[:, 0, :]
    g2 = _gram_diff(u2, rows=2)[:, 0, :]
    g3 = _gram_diff(u3, rows=1)[:, 0, :]

    wsum = {
        0: _warp_sl1_sum(ref, warps[0])[:, 0, :],
        1: _warp_sl1_sum(rr1, warps[1])[:, 0, :],
        2: _warp_sl1_sum(rr2, warps[2])[:, 0, :],
    }

    loss = level_terms[0] + level_terms[1] + level_terms[2]
    sizes = {0: 256, 1: 128, 2: 64}
    cdims = [(64, 224), (128, 112), (256, 56), (512, 28)]
    for i, l in enumerate((2, 1, 0)):
        wl = 2.0 ** (1 - l)
        S = sizes[l]
        loss = loss + wl * (wsum[l][0, 0] + wsum[l][1, 0]) / (2.0 * 3 * S * S)

        V = 0.0
        C, HH = cdims[0]
        V = V + (s0[2 * i, 0] + s0[2 * i + 1, 0]) / (2.0 * C * HH * HH)
        for b, sb, gb, wgt in ((1, s1, g1, 1000.0 / (224.0 * 224.0)),
                               (2, s2, g2, 1000.0 / (112.0 * 112.0))):
            C, HH = cdims[b]
            denom = 2.0 * C * HH * HH
            V = V + wgt * (sb[2 * i, 1] + sb[2 * i + 1, 1]) / denom
            V = V + gb[i, 0] / (denom * denom * (2.0 * C) ** 2)
            V = V + (sb[2 * i, 0] + sb[2 * i + 1, 0]) / denom
        C, HH = cdims[3]
        denom = 2.0 * C * HH * HH
        sxg = jnp.concatenate([v3[2 * i, 0], v3[2 * i + 1, 0]]) + 1e-07
        uxg = jnp.concatenate([v3[2 * i, 1], v3[2 * i + 1, 1]]) + 1e-07
        vxg = jnp.concatenate([v3[2 * i, 2], v3[2 * i + 1, 2]]) + 1e-07
        syt = jnp.concatenate([v3[2 * i, 3], v3[2 * i + 1, 3]]) + 1e-07
        uyt = jnp.concatenate([v3[2 * i, 4], v3[2 * i + 1, 4]]) + 1e-07
        vyt = jnp.concatenate([v3[2 * i, 5], v3[2 * i + 1, 5]]) + 1e-07
        cuk = uyt / syt
        cvk = vyt / syt
        cukp = uxg / sxg
        cvkp = vxg / sxg
        align = (jnp.sum((cuk - cukp) ** 2) + jnp.sum((cvk - cvkp) ** 2)) \
            / (2.0 * cuk.shape[0])
        V = V + align
        V = V + g3[i, 0] / (denom * denom * (2.0 * C) ** 2)
        V = V + (s3[2 * i, 0] + s3[2 * i + 1, 0]) / denom
        loss = loss + wl * V

    return loss


# speedup vs baseline: 1.0144x; 1.0009x over previous
import functools
import math

import jax
import jax.numpy as jnp
import numpy as np
from jax.experimental import pallas as pl
from jax.experimental.pallas import tpu as pltpu

F32 = jnp.float32
BF16 = jnp.bfloat16
PREC = jax.lax.Precision.HIGHEST
CPREC = jax.lax.Precision.DEFAULT
_CP = dict(compiler_params=pltpu.CompilerParams(
    vmem_limit_bytes=100 * 1024 * 1024))


def _resize_mat(T, S, align_corners):
    if align_corners:
        xs = np.linspace(np.float32(0.0), np.float32(S - 1), T).astype(np.float32)
    else:
        xs = np.maximum(
            (np.arange(T, dtype=np.float32) + np.float32(0.5))
            * np.float32(S / T) - np.float32(0.5), np.float32(0.0))
    x0 = np.clip(np.floor(xs).astype(np.int64), 0, S - 1)
    x1 = np.clip(x0 + 1, 0, S - 1)
    w = (xs - x0.astype(np.float32)).astype(np.float32)
    M = np.zeros((T, S), dtype=np.float32)
    M[np.arange(T), x0] += (1.0 - w)
    M[np.arange(T), x1] += w
    return M


def _pool_mat(T):
    M = np.zeros((T, 2 * T), dtype=np.float32)
    M[np.arange(T), 2 * np.arange(T)] = 0.5
    M[np.arange(T), 2 * np.arange(T) + 1] = 0.5
    return M


_R128 = _resize_mat(128, 256, True)
_R64 = _resize_mat(64, 256, True)
_T224_256 = _resize_mat(224, 256, False)
_T224_128 = _resize_mat(224, 128, False)
_T224_64 = _resize_mat(224, 64, False)
_C128 = (_T224_128 @ _R128).astype(np.float32)
_C64 = (_T224_64 @ _R64).astype(np.float32)
_P112 = _pool_mat(112)
_P56 = _pool_mat(56)


def _resize_body(x_ref, a_ref, b_ref, o_ref):
    x = x_ref[0, 0]
    t = jax.lax.dot(a_ref[...], x, preferred_element_type=F32, precision=PREC)
    o_ref[0, 0] = jax.lax.dot(t, b_ref[...], preferred_element_type=F32,
                              precision=PREC)


def _resize_multi_body(x_ref, a_ref, b_ref, o_ref):
    x = x_ref[0, 0]
    t = jax.lax.dot(a_ref[0], x, preferred_element_type=F32, precision=PREC)
    o_ref[0, 0, 0] = jax.lax.dot(t, b_ref[0], preferred_element_type=F32,
                                 precision=PREC)


def _resize_ref5(x, mats):
    A = jnp.asarray(np.stack(mats))
    B = jnp.asarray(np.stack([m.T.copy() for m in mats]))
    return pl.pallas_call(
        _resize_multi_body,
        grid=(2, 3, 5),
        in_specs=[
            pl.BlockSpec((1, 1, 256, 256), lambda n, c, m: (n, c, 0, 0)),
            pl.BlockSpec((1, 224, 256), lambda n, c, m: (m, 0, 0)),
            pl.BlockSpec((1, 256, 224), lambda n, c, m: (m, 0, 0)),
        ],
        out_specs=pl.BlockSpec((1, 1, 1, 224, 224),
                               lambda n, c, m: (n, c, m, 0, 0)),
        out_shape=jax.ShapeDtypeStruct((2, 3, 5, 224, 224), F32),
        **_CP,
    )(x, A, B)


def _resize_warp3(x, mats):
    A = jnp.asarray(np.stack(mats))
    B = jnp.asarray(np.stack([m.T.copy() for m in mats]))
    return pl.pallas_call(
        _resize_multi_body,
        grid=(6, 3),
        in_specs=[
            pl.BlockSpec((1, 1, 256, 256), lambda n, c: (n, c, 0, 0)),
            pl.BlockSpec((1, 224, 256), lambda n, c: (n // 2, 0, 0)),
            pl.BlockSpec((1, 256, 224), lambda n, c: (n // 2, 0, 0)),
        ],
        out_specs=pl.BlockSpec((1, 1, 1, 224, 224),
                               lambda n, c: (n, c, 0, 0, 0)),
        out_shape=jax.ShapeDtypeStruct((6, 3, 1, 224, 224), F32),
        **_CP,
    )(x, A, B)


def _resize(x, A):
    N, C, S, _ = x.shape
    T = A.shape[0]
    Aj = jnp.asarray(A)
    Bj = jnp.asarray(A.T.copy())
    return pl.pallas_call(
        _resize_body,
        grid=(N, C),
        in_specs=[
            pl.BlockSpec((1, 1, S, S), lambda n, c: (n, c, 0, 0)),
            pl.BlockSpec((T, S), lambda n, c: (0, 0)),
            pl.BlockSpec((S, T), lambda n, c: (0, 0)),
        ],
        out_specs=pl.BlockSpec((1, 1, T, T), lambda n, c: (n, c, 0, 0)),
        out_shape=jax.ShapeDtypeStruct((N, C, T, T), F32),
        **_CP,
    )(x, Aj, Bj)


def _halo_rows(x, rows):
    N, H, Wp, C = x.shape
    Ht = H // rows
    z = jnp.zeros((N, 1, Wp, C), x.dtype)
    if rows == 1:
        tops = z
        bots = z
    else:
        tops = jnp.concatenate([z, x[:, Ht - 1:H - 1:Ht]], axis=1)
        bots = jnp.concatenate([x[:, Ht:H:Ht], z], axis=1)
    return jnp.stack([tops, bots], axis=2)


def _conv_body(xm_ref, xb_ref, w_ref, b_ref, o_ref, *, Ht, W, Cin, Cout):
    full = jnp.concatenate(
        [xb_ref[0, 0, 0][None], xm_ref[0], xb_ref[0, 0, 1][None]],
        axis=0)
    acc = jnp.zeros((Ht * W, Cout), dtype=F32)
    for dy in range(3):
        for dx in range(3):
            a2 = full[dy:dy + Ht, dx:dx + W, :].reshape(Ht * W, Cin)
            acc = acc + jax.lax.dot(
                a2, w_ref[dy * 3 + dx], preferred_element_type=F32,
                precision=CPREC)
    y = jnp.maximum(acc + b_ref[...], 0.0).reshape(Ht, W, Cout)
    o_ref[0, :, 1:W + 1, :] = y
    o_ref[0, :, 0:1, :] = jnp.zeros((Ht, 1, Cout), dtype=F32)
    o_ref[0, :, W + 1:W + 2, :] = jnp.zeros((Ht, 1, Cout), dtype=F32)


def _conv(x, w, b, *, rows):
    N, H, Wp, Cin = x.shape
    W = Wp - 2
    Ht = H // rows
    Cout = w.shape[0]
    wt = jnp.transpose(w, (2, 3, 1, 0)).reshape(9, Cin, Cout)
    xb = _halo_rows(x, rows)
    body = functools.partial(_conv_body, Ht=Ht, W=W, Cin=Cin, Cout=Cout)
    return pl.pallas_call(
        body,
        grid=(N, rows),
        in_specs=[
            pl.BlockSpec((1, Ht, Wp, Cin), lambda n, r: (n, r, 0, 0)),
            pl.BlockSpec((1, 1, 2, Wp, Cin), lambda n, r: (n, r, 0, 0, 0)),
            pl.BlockSpec((9, Cin, Cout), lambda n, r: (0, 0, 0)),
            pl.BlockSpec((1, Cout), lambda n, r: (0, 0)),
        ],
        out_specs=pl.BlockSpec((1, Ht, Wp, Cout), lambda n, r: (n, r, 0, 0)),
        out_shape=jax.ShapeDtypeStruct((N, H, Wp, Cout), F32),
        compiler_params=pltpu.CompilerParams(
            dimension_semantics=("parallel", "parallel"),
            vmem_limit_bytes=100 * 1024 * 1024),
    )(x, xb, wt, b.reshape(1, Cout))


def _convp_body(xm_ref, xb_ref, w_ref, b_ref, o_ref, *, Ht, W, Cin, Cout):
    Wp = W + 2
    full = jnp.concatenate(
        [xb_ref[0, 0, 0][None], xm_ref[0], xb_ref[0, 0, 1][None]],
        axis=0)
    a3 = jnp.concatenate(
        [full[0:Ht], full[1:Ht + 1], full[2:Ht + 2]], axis=-1)
    out = jax.lax.dot(a3.reshape(Ht * Wp, 3 * Cin), w_ref[...],
                      preferred_element_type=F32, precision=CPREC)
    out = out.reshape(Ht, Wp, 3 * Cout)
    y = (out[:, 0:W, 0:Cout] + out[:, 1:W + 1, Cout:2 * Cout]
         + out[:, 2:W + 2, 2 * Cout:3 * Cout])
    y = jnp.maximum(y + b_ref[0][None, None, :], 0.0)
    o_ref[0, :, 1:W + 1, :] = y
    o_ref[0, :, 0:1, :] = jnp.zeros((Ht, 1, Cout), dtype=F32)
    o_ref[0, :, W + 1:W + 2, :] = jnp.zeros((Ht, 1, Cout), dtype=F32)


def _convp(x, w, b, *, rows):
    N, H, Wp, Cin = x.shape
    W = Wp - 2
    Ht = H // rows
    Cout = w.shape[0]
    wt = jnp.transpose(w, (2, 1, 3, 0)).reshape(3 * Cin, 3 * Cout)
    xb = _halo_rows(x, rows)
    body = functools.partial(_convp_body, Ht=Ht, W=W, Cin=Cin, Cout=Cout)
    return pl.pallas_call(
        body,
        grid=(N, rows),
        in_specs=[
            pl.BlockSpec((1, Ht, Wp, Cin), lambda n, r: (n, r, 0, 0)),
            pl.BlockSpec((1, 1, 2, Wp, Cin), lambda n, r: (n, r, 0, 0, 0)),
            pl.BlockSpec((3 * Cin, 3 * Cout), lambda n, r: (0, 0)),
            pl.BlockSpec((1, Cout), lambda n, r: (0, 0)),
        ],
        out_specs=pl.BlockSpec((1, Ht, Wp, Cout), lambda n, r: (n, r, 0, 0)),
        out_shape=jax.ShapeDtypeStruct((N, H, Wp, Cout), F32),
        compiler_params=pltpu.CompilerParams(
            dimension_semantics=("parallel", "parallel"),
            vmem_limit_bytes=100 * 1024 * 1024),
    )(x, xb, wt, b.reshape(1, Cout))


def _conv0_body(xm_ref, xb_ref, w_ref, b_ref, o_ref, *, Ht, W, Cout):
    full = jnp.concatenate(
        [xb_ref[0, 0, 0][None], xm_ref[0], xb_ref[0, 0, 1][None]],
        axis=0)
    taps = [full[dy:dy + Ht, dx:dx + W, :]
            for dy in range(3) for dx in range(3)]
    a27 = jnp.concatenate(taps, axis=-1).reshape(Ht * W, 27)
    acc = jax.lax.dot(a27, w_ref[...], preferred_element_type=F32,
                      precision=CPREC)
    y = jnp.maximum(acc + b_ref[...], 0.0).reshape(Ht, W, Cout)
    o_ref[0, :, 1:W + 1, :] = y
    o_ref[0, :, 0:1, :] = jnp.zeros((Ht, 1, Cout), dtype=F32)
    o_ref[0, :, W + 1:W + 2, :] = jnp.zeros((Ht, 1, Cout), dtype=F32)


def _conv0(x, w, b, *, rows):
    N, H, Wp, _ = x.shape
    W = Wp - 2
    Ht = H // rows
    Cout = w.shape[0]
    wt = jnp.transpose(w, (2, 3, 1, 0)).reshape(27, Cout)
    xb = _halo_rows(x, rows)
    body = functools.partial(_conv0_body, Ht=Ht, W=W, Cout=Cout)
    return pl.pallas_call(
        body,
        grid=(N, rows),
        in_specs=[
            pl.BlockSpec((1, Ht, Wp, 3), lambda n, r: (n, r, 0, 0)),
            pl.BlockSpec((1, 1, 2, Wp, 3), lambda n, r: (n, r, 0, 0, 0)),
            pl.BlockSpec((27, Cout), lambda n, r: (0, 0)),
            pl.BlockSpec((1, Cout), lambda n, r: (0, 0)),
        ],
        out_specs=pl.BlockSpec((1, Ht, Wp, Cout), lambda n, r: (n, r, 0, 0)),
        out_shape=jax.ShapeDtypeStruct((N, H, Wp, Cout), F32),
        compiler_params=pltpu.CompilerParams(
            dimension_semantics=("parallel", "parallel"),
            vmem_limit_bytes=100 * 1024 * 1024),
    )(x, xb, wt, b.reshape(1, Cout))


def _pool_body(x_ref, o_ref, *, Ht, W, C):
    a = x_ref[0, :, 1:W + 1, :]
    a = a.reshape(Ht, 2, W, C)
    a = jnp.max(a, axis=1)
    a = a.reshape(Ht, W // 2, 2, C)
    a = jnp.max(a, axis=2)
    w = W // 2
    o_ref[0, :, 1:w + 1, :] = a
    o_ref[0, :, 0:1, :] = jnp.zeros((Ht, 1, C), dtype=F32)
    o_ref[0, :, w + 1:w + 2, :] = jnp.zeros((Ht, 1, C), dtype=F32)


def _maxpool(x, *, rows):
    N, H, Wp, C = x.shape
    W = Wp - 2
    h, w = H // 2, W // 2
    Ht = h // rows
    body = functools.partial(_pool_body, Ht=Ht, W=W, C=C)
    return pl.pallas_call(
        body,
        grid=(N, rows),
        in_specs=[pl.BlockSpec((1, 2 * Ht, Wp, C), lambda n, r: (n, r, 0, 0))],
        out_specs=pl.BlockSpec((1, Ht, w + 2, C), lambda n, r: (n, r, 0, 0)),
        out_shape=jax.ShapeDtypeStruct((N, h, w + 2, C), F32),
        compiler_params=pltpu.CompilerParams(
            dimension_semantics=("parallel", "parallel"),
            vmem_limit_bytes=100 * 1024 * 1024),
    )(x)


def _gm_body(x_ref, p1_ref, p2_ref, g1_ref, g2_ref):
    es = []
    for n in range(2):
        e = ((x_ref[n, 0] - x_ref[n + 2, 0]) ** 2
             + (x_ref[n, 1] - x_ref[n + 2, 1]) ** 2
             + (x_ref[n, 2] - x_ref[n + 2, 2]) ** 2) / 3.0
        es.append(e)
    mn = jnp.minimum(jnp.min(es[0]), jnp.min(es[1]))
    mx = jnp.maximum(jnp.max(es[0]), jnp.max(es[1]))
    scale = 1.0 / (mx - mn)
    p1 = p1_ref[...]
    p2 = p2_ref[...]
    for n in range(2):
        g = (es[n] - mn) * scale
        t = jax.lax.dot(p1, g, preferred_element_type=F32, precision=PREC)
        g112 = jax.lax.dot(t, p1.T, preferred_element_type=F32,
                           precision=PREC)
        g1_ref[0, n] = g112
        t2 = jax.lax.dot(p2, g112, preferred_element_type=F32, precision=PREC)
        g2_ref[0, n] = jax.lax.dot(t2, p2.T, preferred_element_type=F32,
                                   precision=PREC)


def _gm_maps(xp):
    p1 = jnp.asarray(_P112)
    p2 = jnp.asarray(_P56)
    return pl.pallas_call(
        _gm_body,
        grid=(3,),
        in_specs=[
            pl.BlockSpec((4, 3, 224, 224), lambda i: (i, 0, 0, 0)),
            pl.BlockSpec((112, 224), lambda i: (0, 0)),
            pl.BlockSpec((56, 112), lambda i: (0, 0)),
        ],
        out_specs=[
            pl.BlockSpec((1, 2, 112, 112), lambda i: (i, 0, 0, 0)),
            pl.BlockSpec((1, 2, 56, 56), lambda i: (i, 0, 0, 0)),
        ],
        out_shape=[
            jax.ShapeDtypeStruct((3, 2, 112, 112), F32),
            jax.ShapeDtypeStruct((3, 2, 56, 56), F32),
        ],
        compiler_params=pltpu.CompilerParams(
            dimension_semantics=("parallel",),
            vmem_limit_bytes=100 * 1024 * 1024),
    )(xp, p1, p2)


def _make_stats(U, gm=None, align=False, rows=1):
    N, H, Wp, C = U.shape
    W = Wp - 2
    Ht = H // rows

    if align:
        def body(x_ref, y_ref, o_ref, v_ref):
            x = x_ref[0, :, 1:W + 1, :].astype(F32)
            y = y_ref[0, :, 1:W + 1, :].astype(F32)
            d = x - y
            o_ref[0, 0, 0] = jnp.sum(d * d)
            cx = (jax.lax.broadcasted_iota(jnp.int32, (H, W, C), 1)
                  .astype(F32) / 14.0 - 1.0)
            cy = (jax.lax.broadcasted_iota(jnp.int32, (H, W, C), 0)
                  .astype(F32) / 14.0 - 1.0)
            v_ref[0, 0] = jnp.sum(x, axis=(0, 1))
            v_ref[0, 1] = jnp.sum(cx * x, axis=(0, 1))
            v_ref[0, 2] = jnp.sum(cy * x, axis=(0, 1))
            v_ref[0, 3] = jnp.sum(y, axis=(0, 1))
            v_ref[0, 4] = jnp.sum(cx * y, axis=(0, 1))
            v_ref[0, 5] = jnp.sum(cy * y, axis=(0, 1))

        out_specs = [
            pl.BlockSpec((1, 1, 2), lambda s, r: (s, 0, 0),
                         memory_space=pltpu.SMEM),
            pl.BlockSpec((1, 8, C), lambda s, r: (s, 0, 0)),
        ]
        out_shape = [
            jax.ShapeDtypeStruct((6, 1, 2), F32),
            jax.ShapeDtypeStruct((6, 8, C), F32),
        ]
        extra_in = []
    elif gm is not None:
        def body(x_ref, y_ref, g_ref, o_ref):
            r = pl.program_id(1)

            @pl.when(r == 0)
            def _():
                o_ref[0, 0, 0] = 0.0
                o_ref[0, 0, 1] = 0.0
            d = (x_ref[0, :, 1:W + 1, :].astype(F32)
                 - y_ref[0, :, 1:W + 1, :].astype(F32))
            d2 = d * d
            g = g_ref[0, 0][:, :, None]
            o_ref[0, 0, 0] += jnp.sum(d2)
            o_ref[0, 0, 1] += jnp.sum((g * g) * d2)

        out_specs = [
            pl.BlockSpec((1, 1, 2), lambda s, r: (s, 0, 0),
                         memory_space=pltpu.SMEM)]
        out_shape = [jax.ShapeDtypeStruct((6, 1, 2), F32)]
        extra_in = [
            pl.BlockSpec((1, 1, Ht, W),
                         lambda s, r: (s // 2, s % 2, r, 0))]
    else:
        def body(x_ref, y_ref, o_ref):
            r = pl.program_id(1)

            @pl.when(r == 0)
            def _():
                o_ref[0, 0, 0] = 0.0
            d = (x_ref[0, :, 1:W + 1, :].astype(F32)
                 - y_ref[0, :, 1:W + 1, :].astype(F32))
            o_ref[0, 0, 0] += jnp.sum(d * d)

        out_specs = [
            pl.BlockSpec((1, 1, 2), lambda s, r: (s, 0, 0),
                         memory_space=pltpu.SMEM)]
        out_shape = [jax.ShapeDtypeStruct((6, 1, 2), F32)]
        extra_in = []

    in_specs = [
        pl.BlockSpec((1, Ht, Wp, C),
                     lambda s, r: (4 * (s // 2) + s % 2, r, 0, 0)),
        pl.BlockSpec((1, Ht, Wp, C),
                     lambda s, r: (4 * (s // 2) + s % 2 + 2, r, 0, 0)),
    ] + extra_in

    args = [U, U] + ([gm] if gm is not None else [])
    return pl.pallas_call(
        body,
        grid=(6, rows),
        in_specs=in_specs,
        out_specs=out_specs if len(out_specs) > 1 else out_specs[0],
        out_shape=out_shape if len(out_shape) > 1 else out_shape[0],
        compiler_params=pltpu.CompilerParams(
            dimension_semantics=("parallel", "arbitrary"),
            vmem_limit_bytes=100 * 1024 * 1024),
    )(*args)


def _gram_body(x0_ref, x1_ref, y0_ref, y1_ref, o_ref, acc_ref, *,
               C, W, HWt, rows):
    r = pl.program_id(1)

    @pl.when(r == 0)
    def _():
        acc_ref[...] = jnp.zeros(acc_ref.shape, acc_ref.dtype)

    def blocks(a_ref, b_ref):
        a = a_ref[0, :, 1:W + 1, :].reshape(HWt, C)
        b = b_ref[0, :, 1:W + 1, :].reshape(HWt, C)
        dn = (((0,), (0,)), ((), ()))
        aa = jax.lax.dot_general(a, a, dn, preferred_element_type=F32,
                                 precision=CPREC)
        ab = jax.lax.dot_general(a, b, dn, preferred_element_type=F32,
                                 precision=CPREC)
        bb = jax.lax.dot_general(b, b, dn, preferred_element_type=F32,
                                 precision=CPREC)
        return aa, ab, bb

    xa, xb, xc = blocks(x0_ref, x1_ref)
    ya, yb, yc = blocks(y0_ref, y1_ref)
    acc_ref[0] += xa - ya
    acc_ref[1] += xb - yb
    acc_ref[2] += xc - yc

    @pl.when(r == rows - 1)
    def _():
        d0 = acc_ref[0]
        d1 = acc_ref[1]
        d2 = acc_ref[2]
        o_ref[0, 0, 0] = (jnp.sum(d0 * d0) + 2.0 * jnp.sum(d1 * d1)
                          + jnp.sum(d2 * d2))


def _gram_diff(U, rows):
    N, H, Wp, C = U.shape
    W = Wp - 2
    Ht = H // rows
    body = functools.partial(_gram_body, C=C, W=W, HWt=Ht * W, rows=rows)

    def mk(img_off):
        return pl.BlockSpec(
            (1, Ht, Wp, C), lambda i, r, o=img_off: (4 * i + o, r, 0, 0))

    return pl.pallas_call(
        body,
        grid=(3, rows),
        in_specs=[mk(0), mk(1), mk(2), mk(3)],
        out_specs=pl.BlockSpec((1, 1, 2), lambda i, r: (i, 0, 0),
                               memory_space=pltpu.SMEM),
        out_shape=jax.ShapeDtypeStruct((3, 1, 2), F32),
        scratch_shapes=[pltpu.VMEM((3, C, C), F32)],
        compiler_params=pltpu.CompilerParams(
            dimension_semantics=("parallel", "arbitrary"),
            vmem_limit_bytes=100 * 1024 * 1024),
    )(U, U, U, U)


def _sl1(d):
    a = jnp.abs(d)
    return jnp.where(a < 1.0, 0.5 * a * a, a - 0.5)


def _level_body(dp_ref, tp_ref, m_ref, v_ref, o_ref, *, H, W):
    q = dp_ref[0]
    p = tp_ref[0]
    v = v_ref[0]

    def grads(z):
        gy = (z[0:H, 0:W] + 2.0 * z[0:H, 1:W + 1] + z[0:H, 2:W + 2]
              - z[2:H + 2, 0:W] - 2.0 * z[2:H + 2, 1:W + 1]
              - z[2:H + 2, 2:W + 2])
        gx = (z[0:H, 0:W] + 2.0 * z[1:H + 1, 0:W] + z[2:H + 2, 0:W]
              - z[0:H, 2:W + 2] - 2.0 * z[1:H + 1, 2:W + 2]
              - z[2:H + 2, 2:W + 2])
        return gy, gx

    gyf, gxf = grads(q)
    gyr, gxr = grads(p)
    gyf = gyf * v
    gxf = gxf * v
    gyr = gyr * v
    gxr = gxr * v
    o_ref[0, 0, 0] = jnp.sum(jnp.abs(gyr - gyf)) + jnp.sum(jnp.abs(gxr - gxf))
    o_ref[0, 0, 1] = jnp.sum(gyf * gyr)
    o_ref[0, 0, 2] = jnp.sum(gxf * gxr)
    o_ref[0, 0, 3] = jnp.sum(gyf * gyf)
    o_ref[0, 0, 4] = jnp.sum(gxf * gxf)
    o_ref[0, 0, 5] = jnp.sum(gyr * gyr)
    o_ref[0, 0, 6] = jnp.sum(gxr * gxr)
    m = m_ref[0]
    o_ref[0, 0, 7] = jnp.sum(_sl1(q[1:H + 1, 1:W + 1] - p[1:H + 1, 1:W + 1]) * m)
    o_ref[0, 0, 8] = jnp.sum(m)


def _level_stats(dpad, tpad, m, valid):
    N, Hp, Wp = dpad.shape
    H, W = Hp - 2, Wp - 2
    body = functools.partial(_level_body, H=H, W=W)
    return pl.pallas_call(
        body,
        grid=(N,),
        in_specs=[
            pl.BlockSpec((1, Hp, Wp), lambda n: (n, 0, 0)),
            pl.BlockSpec((1, Hp, Wp), lambda n: (n, 0, 0)),
            pl.BlockSpec((1, H, W), lambda n: (n, 0, 0)),
            pl.BlockSpec((1, H, W), lambda n: (n, 0, 0)),
        ],
        out_specs=pl.BlockSpec((1, 1, 16), lambda n: (n, 0, 0),
                               memory_space=pltpu.SMEM),
        out_shape=jax.ShapeDtypeStruct((N, 1, 16), F32),
        compiler_params=pltpu.CompilerParams(
            dimension_semantics=("parallel",),
            vmem_limit_bytes=100 * 1024 * 1024),
    )(dpad, tpad, m, valid)


def _warp_body(a_ref, b_ref, o_ref):
    o_ref[0, 0, 0] = jnp.sum(_sl1(a_ref[0] - b_ref[0]))


def _warp_sl1_sum(a, b):
    N = a.shape[0]
    S = a.shape[2]
    return pl.pallas_call(
        _warp_body,
        grid=(N,),
        in_specs=[
            pl.BlockSpec((1, 3, S, S), lambda n: (n, 0, 0, 0)),
            pl.BlockSpec((1, 3, S, S), lambda n: (n, 0, 0, 0)),
        ],
        out_specs=pl.BlockSpec((1, 1, 2), lambda n: (n, 0, 0),
                               memory_space=pltpu.SMEM),
        out_shape=jax.ShapeDtypeStruct((N, 1, 2), F32),
        compiler_params=pltpu.CompilerParams(
            dimension_semantics=("parallel",),
            vmem_limit_bytes=100 * 1024 * 1024),
    )(a, b)


def kernel(depth_0, depth_1, depth_2, target_level_0, target_level_1,
           target_level_2, mask_level_0, mask_level_1, mask_level_2,
           warp_view_0, warp_view_1, warp_view_2, imgs,
           vgg_w0, vgg_b0, vgg_w2, vgg_b2, vgg_w5, vgg_b5, vgg_w7, vgg_b7,
           vgg_w10, vgg_b10, vgg_w12, vgg_b12, vgg_w14, vgg_b14,
           vgg_w17, vgg_b17, vgg_w19, vgg_b19, vgg_w21, vgg_b21):
    depths = (depth_0, depth_1, depth_2)
    targets = (target_level_0, target_level_1, target_level_2)
    masks = (mask_level_0, mask_level_1, mask_level_2)
    warps = (warp_view_0, warp_view_1, warp_view_2)

    dstk = jnp.concatenate(
        [jnp.pad(depths[l], ((0, 0), (1, 1 + 256 - depths[l].shape[1]),
                             (1, 1 + 256 - depths[l].shape[2])))
         for l in range(3)], axis=0)
    tstk = jnp.concatenate(
        [jnp.pad(targets[l], ((0, 0), (1, 1 + 256 - targets[l].shape[1]),
                              (1, 1 + 256 - targets[l].shape[2])))
         for l in range(3)], axis=0)
    mstk = jnp.concatenate(
        [jnp.pad(masks[l].astype(F32),
                 ((0, 0), (0, 256 - masks[l].shape[1]),
                  (0, 256 - masks[l].shape[2])))
         for l in range(3)], axis=0)
    vstk = jnp.concatenate(
        [jnp.pad(jnp.ones((2,) + depths[l].shape[1:], F32),
                 ((0, 0), (0, 256 - depths[l].shape[1]),
                  (0, 256 - depths[l].shape[2])))
         for l in range(3)], axis=0)
    st_all = _level_stats(dstk, tstk, mstk, vstk)[:, 0, :]
    level_terms = []
    for l in range(3):
        st = st_all[2 * l:2 * l + 2]
        H = depths[l].shape[1]
        W = depths[l].shape[2]
        grad_loss = jnp.sum(st[:, 0]) / (2.0 * 2.0 * H * W)
        cosines = []
        for n in range(2):
            cosines.append(st[n, 1] / (jnp.sqrt(st[n, 3]) * jnp.sqrt(st[n, 5])))
            cosines.append(st[n, 2] / (jnp.sqrt(st[n, 4]) * jnp.sqrt(st[n, 6])))
        n_loss = 1.0 - (cosines[0] + cosines[1] + cosines[2] + cosines[3]) / 4.0
        l1 = jnp.sum(st[:, 7]) / jnp.maximum(jnp.sum(st[:, 8]), 1.0)
        level_terms.append((grad_loss + n_loss + l1) * (2.0 ** (1 - l)))

    ref = imgs[:, 0]
    m5 = [np.zeros((224, 256), np.float32) for _ in range(5)]
    m5[0][:128] = _R128
    m5[1][:64] = _R64
    m5[2][:] = _C64
    m5[3][:] = _C128
    m5[4][:] = _T224_256
    r5 = _resize_ref5(ref, m5)
    rr1 = r5[:, :, 0, :128, :128]
    rr2 = r5[:, :, 1, :64, :64]
    x224 = {2: r5[:, :, 2], 1: r5[:, :, 3], 0: r5[:, :, 4]}

    m3 = [np.zeros((224, 256), np.float32) for _ in range(3)]
    m3[0][:, :64] = _T224_64
    m3[1][:, :128] = _T224_128
    m3[2][:] = _T224_256
    wstk = jnp.concatenate(
        [jnp.pad(warps[l], ((0, 0), (0, 0),
                            (0, 256 - warps[l].shape[2]),
                            (0, 256 - warps[l].shape[3])))
         for l in (2, 1, 0)], axis=0)
    y6 = _resize_warp3(wstk, m3)[:, :, 0]
    y224 = {2: y6[0:2], 1: y6[2:4], 0: y6[4:6]}
    xp = jnp.concatenate([x224[2], y224[2], x224[1], y224[1],
                          x224[0], y224[0]], axis=0)

    gm112, gm56 = _gm_maps(xp)

    xn = jnp.pad(jnp.transpose(xp, (0, 2, 3, 1)),
                 ((0, 0), (0, 0), (1, 1), (0, 0)))

    a = _convp(xn, vgg_w0, vgg_b0, rows=7)
    u0 = _convp(a, vgg_w2, vgg_b2, rows=7)
    a = _maxpool(u0, rows=7)
    a = _convp(a, vgg_w5, vgg_b5, rows=2)
    u1 = _convp(a, vgg_w7, vgg_b7, rows=2)
    a = _maxpool(u1, rows=4)
    a = _conv(a, vgg_w10, vgg_b10, rows=2)
    a = _conv(a, vgg_w12, vgg_b12, rows=2)
    u2 = _conv(a, vgg_w14, vgg_b14, rows=2)
    a = _maxpool(u2, rows=2)
    a = _conv(a, vgg_w17, vgg_b17, rows=1)
    a = _conv(a, vgg_w19, vgg_b19, rows=1)
    u3 = _conv(a, vgg_w21, vgg_b21, rows=1)

    s0 = _make_stats(u0, rows=8)[:, 0, :]
    s1 = _make_stats(u1, gm=gm112, rows=2)[:, 0, :]
    s2 = _make_stats(u2, gm=gm56)[:, 0, :]
    s3, v3 = _make_stats(u3, align=True)
    s3 = s3[:, 0, :]
    g1 = _gram_diff(u1, rows=4)[:, 0, :]
    g2 = _gram_diff(u2, rows=2)[:, 0, :]
    g3 = _gram_diff(u3, rows=1)[:, 0, :]

    astk = jnp.concatenate(
        [jnp.pad(rr2, ((0, 0), (0, 0), (0, 192), (0, 192))),
         jnp.pad(rr1, ((0, 0), (0, 0), (0, 128), (0, 128))),
         ref], axis=0)
    ws = _warp_sl1_sum(astk, wstk)[:, 0, :]
    wsum = {2: ws[0:2], 1: ws[2:4], 0: ws[4:6]}

    loss = level_terms[0] + level_terms[1] + level_terms[2]
    sizes = {0: 256, 1: 128, 2: 64}
    cdims = [(64, 224), (128, 112), (256, 56), (512, 28)]
    for i, l in enumerate((2, 1, 0)):
        wl = 2.0 ** (1 - l)
        S = sizes[l]
        loss = loss + wl * (wsum[l][0, 0] + wsum[l][1, 0]) / (2.0 * 3 * S * S)

        V = 0.0
        C, HH = cdims[0]
        V = V + (s0[2 * i, 0] + s0[2 * i + 1, 0]) / (2.0 * C * HH * HH)
        for b, sb, gb, wgt in ((1, s1, g1, 1000.0 / (224.0 * 224.0)),
                               (2, s2, g2, 1000.0 / (112.0 * 112.0))):
            C, HH = cdims[b]
            denom = 2.0 * C * HH * HH
            V = V + wgt * (sb[2 * i, 1] + sb[2 * i + 1, 1]) / denom
            V = V + gb[i, 0] / (denom * denom * (2.0 * C) ** 2)
            V = V + (sb[2 * i, 0] + sb[2 * i + 1, 0]) / denom
        C, HH = cdims[3]
        denom = 2.0 * C * HH * HH
        sxg = jnp.concatenate([v3[2 * i, 0], v3[2 * i + 1, 0]]) + 1e-07
        uxg = jnp.concatenate([v3[2 * i, 1], v3[2 * i + 1, 1]]) + 1e-07
        vxg = jnp.concatenate([v3[2 * i, 2], v3[2 * i + 1, 2]]) + 1e-07
        syt = jnp.concatenate([v3[2 * i, 3], v3[2 * i + 1, 3]]) + 1e-07
        uyt = jnp.concatenate([v3[2 * i, 4], v3[2 * i + 1, 4]]) + 1e-07
        vyt = jnp.concatenate([v3[2 * i, 5], v3[2 * i + 1, 5]]) + 1e-07
        cuk = uyt / syt
        cvk = vyt / syt
        cukp = uxg / sxg
        cvkp = vxg / sxg
        align = (jnp.sum((cuk - cukp) ** 2) + jnp.sum((cvk - cvkp) ** 2)) \
            / (2.0 * cuk.shape[0])
        V = V + align
        V = V + g3[i, 0] / (denom * denom * (2.0 * C) ** 2)
        V = V + (s3[2 * i, 0] + s3[2 * i + 1, 0]) / denom
        loss = loss + wl * V

    return loss
